# Initial kernel scaffold; baseline (speedup 1.0000x reference)
import jax
import jax.numpy as jnp
from jax.experimental import pallas as pl


def kernel(x, attention_output):
    raise NotImplementedError("write your pallas kernel here")


# speedup vs baseline: 3.2527x; 3.2527x over previous
import jax
import jax.numpy as jnp
from jax.experimental import pallas as pl
from jax.experimental.pallas import tpu as pltpu


_ROWS = 4 * 4096
_K = 128
_C = 2048
_STRIDE = 16
_BM = 512


def _upsample_block(x_ref, o_ref):
    xb = x_ref[...]
    lane = jax.lax.broadcasted_iota(jnp.int32, (_BM, 128), 1)
    keep = (lane % _STRIDE) == 0
    base_idx = lane // _STRIDE
    for a in range(_C // 128):
        g = jnp.take_along_axis(xb, base_idx + 8 * a, axis=1)
        o_ref[:, a * 128:(a + 1) * 128] = jnp.where(keep, g, 0.0)


def kernel(x, attention_output):
    del attention_output
    xf = x.reshape(_ROWS, _K)
    out = pl.pallas_call(
        _upsample_block,
        grid=(_ROWS // _BM,),
        in_specs=[pl.BlockSpec((_BM, _K), lambda i: (i, 0))],
        out_specs=pl.BlockSpec((_BM, _C), lambda i: (i, 0)),
        out_shape=jax.ShapeDtypeStruct((_ROWS, _C), jnp.float32),
    )(xf)
    return out.reshape(4, 4096, _C)
